# Initial kernel scaffold; baseline (speedup 1.0000x reference)
import jax
import jax.numpy as jnp
from jax.experimental import pallas as pl


def kernel(x_piece, edge_index_piece, batch, W1, b1, W2, b2, W3, b3):
    raise NotImplementedError("write your pallas kernel here")


# speedup vs baseline: 11.6056x; 11.6056x over previous
import functools

import jax
import jax.numpy as jnp
from jax import lax
from jax.experimental import pallas as pl
from jax.experimental.pallas import tpu as pltpu
from jax.experimental.pallas import tpu_sc as plsc

N = 10000
E = 320000
D = 128

NC = 2
NS = 16
NW = NC * NS
EPW = E // NW
K = 80
NCHUNK = EPW // K
RSTRIDE = 624
RWIN = 640
ZR = 160
DEGW = 16

BLK = 1000
NBLK = N // BLK

_SC_MESH = dict(core_axis_name="c", subcore_axis_name="s",
                num_cores=NC, num_subcores=NS)


def _zero_rows(zbuf, width):
    z16 = jnp.zeros((16,), jnp.float32)

    def body(i, _):
        for j in range(width // 16):
            zbuf[i, pl.ds(j * 16, 16)] = z16
        return 0

    lax.fori_loop(0, ZR, body, 0)


@functools.cache
def _build_deg_kernel():
    return pl.kernel(
        _deg_body,
        out_type=jax.ShapeDtypeStruct((NC, N, DEGW), jnp.float32),
        mesh=plsc.VectorSubcoreMesh(**_SC_MESH),
        scratch_types=[
            pltpu.VMEM_SHARED((N, DEGW), jnp.float32),
            pltpu.VMEM((K,), jnp.int32),
            pltpu.VMEM((K, DEGW), jnp.float32),
            pltpu.VMEM((ZR, DEGW), jnp.float32),
        ],
    )


def _deg_body(dst_hbm, out_hbm, dacc, dst_v, ones_v, zbuf):
    cid = lax.axis_index("c")
    sid = lax.axis_index("s")
    wid = sid * NC + cid
    base = wid * EPW

    one16 = jnp.ones((16,), jnp.float32)

    def fill(i, _):
        ones_v[i, :] = one16
        return 0

    lax.fori_loop(0, K, fill, 0)
    _zero_rows(zbuf, DEGW)
    for z in range(RWIN // ZR):
        pltpu.sync_copy(zbuf, dacc.at[pl.ds(sid * RSTRIDE + z * ZR, ZR)])
    plsc.subcore_barrier()

    def body(i, _):
        pltpu.sync_copy(dst_hbm.at[pl.ds(base + i * K, K)], dst_v)
        pltpu.sync_copy(ones_v, dacc.at[dst_v], add=True)
        return 0

    lax.fori_loop(0, NCHUNK, body, 0)
    plsc.subcore_barrier()
    pltpu.sync_copy(dacc.at[pl.ds(sid * RSTRIDE, RWIN)],
                    out_hbm.at[cid, pl.ds(sid * RSTRIDE, RWIN)])


@functools.cache
def _build_scatter_kernel():
    return pl.kernel(
        _scatter_body,
        out_type=jax.ShapeDtypeStruct((NC, N, D), jnp.float32),
        mesh=plsc.VectorSubcoreMesh(**_SC_MESH),
        scratch_types=[
            pltpu.VMEM_SHARED((N, D), jnp.float32),
            pltpu.VMEM((K,), jnp.int32),
            pltpu.VMEM((K,), jnp.int32),
            pltpu.VMEM((K, D), jnp.float32),
            pltpu.VMEM((ZR, D), jnp.float32),
            pltpu.SemaphoreType.DMA,
        ],
    )


def _scatter_body(g_hbm, src_hbm, dst_hbm, out_hbm,
                  acc, src_v, dst_v, rows_v, zbuf, sem):
    cid = lax.axis_index("c")
    sid = lax.axis_index("s")
    wid = sid * NC + cid
    base = wid * EPW

    _zero_rows(zbuf, D)
    for z in range(RWIN // ZR):
        pltpu.sync_copy(zbuf, acc.at[pl.ds(sid * RSTRIDE + z * ZR, ZR)])
    plsc.subcore_barrier()

    def body(i, _):
        off = base + i * K
        pltpu.sync_copy(src_hbm.at[pl.ds(off, K)], src_v)
        pltpu.sync_copy(dst_hbm.at[pl.ds(off, K)], dst_v)
        pltpu.async_copy(g_hbm.at[src_v], rows_v, sem).wait()
        pltpu.sync_copy(rows_v, acc.at[dst_v], add=True)
        return 0

    lax.fori_loop(0, NCHUNK, body, 0)
    plsc.subcore_barrier()
    pltpu.sync_copy(acc.at[pl.ds(sid * RSTRIDE, RWIN)],
                    out_hbm.at[cid, pl.ds(sid * RSTRIDE, RWIN)])


def _dis_inv(degp):
    deg = degp[0, :, 0] + degp[1, :, 0] + 1.0
    return lax.rsqrt(deg), 1.0 / deg


def _gelu(x):
    return 0.5 * x * (1.0 + lax.erf(x * (2.0 ** -0.5)))


def _tc1_body(x_ref, w_ref, degp_ref, h_ref, g_ref):
    dis, _ = _dis_inv(degp_ref[...])
    h = jnp.dot(x_ref[...], w_ref[...], preferred_element_type=jnp.float32)
    h_ref[...] = h
    g_ref[...] = h * dis[:, None]


def _tc_mid_body(sp_ref, hp_ref, degp_ref, b_ref, w_ref, h_ref, g_ref):
    dis, inv = _dis_inv(degp_ref[...])
    s = sp_ref[0] + sp_ref[1]
    pre = s * dis[:, None] + hp_ref[...] * inv[:, None] + b_ref[...]
    a = _gelu(pre)
    h = jnp.dot(a, w_ref[...], preferred_element_type=jnp.float32)
    h_ref[...] = h
    g_ref[...] = h * dis[:, None]


def _tc_fin_body(sp_ref, hp_ref, degp_ref, b_ref, out_ref):
    dis, inv = _dis_inv(degp_ref[...])
    s = sp_ref[0] + sp_ref[1]
    out_ref[...] = s * dis[:, None] + hp_ref[...] * inv[:, None] + b_ref[...]


_ROW_SPEC = pl.BlockSpec((BLK, D), lambda i: (i, 0))
_PART_SPEC = pl.BlockSpec((NC, BLK, D), lambda i: (0, i, 0))
_DEGP_SPEC = pl.BlockSpec((NC, BLK, DEGW), lambda i: (0, i, 0))
_W_SPEC = pl.BlockSpec((D, D), lambda i: (0, 0))
_B_SPEC = pl.BlockSpec((1, D), lambda i: (0, 0))
_ND_F32 = jax.ShapeDtypeStruct((N, D), jnp.float32)


def _tc1(x, w, degp):
    return pl.pallas_call(
        _tc1_body,
        grid=(NBLK,),
        in_specs=[_ROW_SPEC, _W_SPEC, _DEGP_SPEC],
        out_specs=[_ROW_SPEC, _ROW_SPEC],
        out_shape=[_ND_F32, _ND_F32],
    )(x, w, degp)


def _tc_mid(sp, hp, degp, b, w):
    return pl.pallas_call(
        _tc_mid_body,
        grid=(NBLK,),
        in_specs=[_PART_SPEC, _ROW_SPEC, _DEGP_SPEC, _B_SPEC, _W_SPEC],
        out_specs=[_ROW_SPEC, _ROW_SPEC],
        out_shape=[_ND_F32, _ND_F32],
    )(sp, hp, degp, b.reshape(1, D), w)


def _tc_fin(sp, hp, degp, b):
    return pl.pallas_call(
        _tc_fin_body,
        grid=(NBLK,),
        in_specs=[_PART_SPEC, _ROW_SPEC, _DEGP_SPEC, _B_SPEC],
        out_specs=_ROW_SPEC,
        out_shape=_ND_F32,
    )(sp, hp, degp, b.reshape(1, D))


def kernel(x_piece, edge_index_piece, batch, W1, b1, W2, b2, W3, b3):
    ei = edge_index_piece.astype(jnp.int32)
    src, dst = ei[0], ei[1]

    deg_k = _build_deg_kernel()
    scat_k = _build_scatter_kernel()
    degp = deg_k(dst)

    h1, g1 = _tc1(x_piece, W1, degp)
    s1 = scat_k(g1, src, dst)
    h2, g2 = _tc_mid(s1, h1, degp, b1, W2)
    s2 = scat_k(g2, src, dst)
    h3, g3 = _tc_mid(s2, h2, degp, b2, W3)
    s3 = scat_k(g3, src, dst)
    return _tc_fin(s3, h3, degp, b3)


# speedup vs baseline: 30.3612x; 2.6161x over previous
import functools

import jax
import jax.numpy as jnp
from jax import lax
from jax.experimental import pallas as pl
from jax.experimental.pallas import tpu as pltpu
from jax.experimental.pallas import tpu_sc as plsc

N = 10000
E = 320000
D = 128

NC = 2
NS = 16
NW = NC * NS
K = 128
NCH = E // K
CHW_LO = NCH // NW
CHW_EXTRA = NCH - CHW_LO * NW

RSTRIDE = 624
RWIN = 640
ZR = 128
DEGW = 16

BLK = 1000
NBLK = N // BLK

_SC_MESH = dict(core_axis_name="c", subcore_axis_name="s",
                num_cores=NC, num_subcores=NS)


def _worker_chunks(wid):
    extra = jnp.minimum(wid, CHW_EXTRA)
    start = wid * CHW_LO + extra
    n = jnp.where(wid < CHW_EXTRA, CHW_LO + 1, CHW_LO)
    return start, n


def _zero_rows(zbuf, nrows, width):
    z16 = jnp.zeros((16,), jnp.float32)

    def body(i, _):
        for j in range(width // 16):
            zbuf[i, pl.ds(j * 16, 16)] = z16
        return 0

    lax.fori_loop(0, nrows, body, 0)


@functools.cache
def _build_deg_kernel():
    return pl.kernel(
        _deg_body,
        out_type=jax.ShapeDtypeStruct((NC, N, DEGW), jnp.float32),
        mesh=plsc.VectorSubcoreMesh(**_SC_MESH),
        scratch_types=[
            pltpu.VMEM_SHARED((N, DEGW), jnp.float32),
            pltpu.VMEM((2, K), jnp.int32),
            pltpu.VMEM((2, K), jnp.int32),
            pltpu.VMEM((2, K), jnp.int32),
            pltpu.VMEM((K, DEGW), jnp.float32),
            pltpu.VMEM((ZR, DEGW), jnp.float32),
            pltpu.SemaphoreType.DMA,
            pltpu.SemaphoreType.DMA,
            pltpu.SemaphoreType.DMA,
            pltpu.SemaphoreType.DMA,
            pltpu.SemaphoreType.DMA,
            pltpu.SemaphoreType.DMA,
        ],
    )


def _deg_body(ei_hbm, out_hbm, dacc, i0, i1, i2, ones_v, zbuf,
              si0, si1, si2, ss0, ss1, ss2):
    cid = lax.axis_index("c")
    sid = lax.axis_index("s")
    wid = sid * NC + cid
    ch0, ncw = _worker_chunks(wid)

    idx = (i0, i1, i2)
    sem_i = (si0, si1, si2)
    sem_s = (ss0, ss1, ss2)

    one16 = jnp.ones((16,), jnp.float32)

    def fill(i, _):
        ones_v[i, :] = one16
        return 0

    lax.fori_loop(0, K, fill, 0)
    _zero_rows(zbuf, ZR, DEGW)
    for z in range(RWIN // ZR):
        pltpu.sync_copy(zbuf, dacc.at[pl.ds(sid * RSTRIDE + z * ZR, ZR)])
    plsc.subcore_barrier()

    def outer(t, _):
        for k in range(3):
            i = t * 3 + k
            a, b = k, (k - 1) % 3

            @pl.when((i >= 3) & (i < ncw + 3))
            def _():
                pltpu.make_async_copy(ones_v, dacc.at[idx[a].at[1]],
                                      sem_s[a]).wait()

            @pl.when(i < ncw)
            def _():
                off = (ch0 + i) * K
                pltpu.async_copy(ei_hbm.at[:, pl.ds(off, K)], idx[a],
                                 sem_i[a])

            @pl.when((i >= 1) & (i <= ncw))
            def _():
                pltpu.make_async_copy(ei_hbm.at[:, pl.ds(0, K)], idx[b],
                                      sem_i[b]).wait()
                pltpu.async_copy(ones_v, dacc.at[idx[b].at[1]], sem_s[b],
                                 add=True)

        return 0

    lax.fori_loop(0, (ncw + 5) // 3, outer, 0)
    plsc.subcore_barrier()
    pltpu.sync_copy(dacc.at[pl.ds(sid * RSTRIDE, RWIN)],
                    out_hbm.at[cid, pl.ds(sid * RSTRIDE, RWIN)])


@functools.cache
def _build_scatter_kernel():
    return pl.kernel(
        _scatter_body,
        out_type=jax.ShapeDtypeStruct((NC, N, D), jnp.float32),
        mesh=plsc.VectorSubcoreMesh(**_SC_MESH),
        scratch_types=[
            pltpu.VMEM_SHARED((N, D), jnp.float32),
            pltpu.VMEM((2, K), jnp.int32),
            pltpu.VMEM((2, K), jnp.int32),
            pltpu.VMEM((2, K), jnp.int32),
            pltpu.VMEM((K, D), jnp.float32),
            pltpu.VMEM((K, D), jnp.float32),
            pltpu.VMEM((K, D), jnp.float32),
            pltpu.SemaphoreType.DMA,
            pltpu.SemaphoreType.DMA,
            pltpu.SemaphoreType.DMA,
            pltpu.SemaphoreType.DMA,
            pltpu.SemaphoreType.DMA,
            pltpu.SemaphoreType.DMA,
            pltpu.SemaphoreType.DMA,
            pltpu.SemaphoreType.DMA,
            pltpu.SemaphoreType.DMA,
        ],
    )


def _scatter_body(g_hbm, ei_hbm, out_hbm, acc, i0, i1, i2, r0, r1, r2,
                  si0, si1, si2, sg0, sg1, sg2, ss0, ss1, ss2):
    cid = lax.axis_index("c")
    sid = lax.axis_index("s")
    wid = sid * NC + cid
    ch0, ncw = _worker_chunks(wid)

    idx = (i0, i1, i2)
    rows = (r0, r1, r2)
    sem_i = (si0, si1, si2)
    sem_g = (sg0, sg1, sg2)
    sem_s = (ss0, ss1, ss2)

    _zero_rows(r0, ZR, D)
    for z in range(RWIN // ZR):
        pltpu.sync_copy(r0, acc.at[pl.ds(sid * RSTRIDE + z * ZR, ZR)])
    plsc.subcore_barrier()

    def outer(t, _):
        for k in range(3):
            i = t * 3 + k
            a, b, c = k, (k - 1) % 3, (k - 2) % 3

            @pl.when((i >= 3) & (i < ncw + 3))
            def _():
                pltpu.make_async_copy(rows[a], acc.at[idx[a].at[1]],
                                      sem_s[a]).wait()

            @pl.when(i < ncw)
            def _():
                off = (ch0 + i) * K
                pltpu.async_copy(ei_hbm.at[:, pl.ds(off, K)], idx[a],
                                 sem_i[a])

            @pl.when((i >= 1) & (i <= ncw))
            def _():
                pltpu.make_async_copy(ei_hbm.at[:, pl.ds(0, K)], idx[b],
                                      sem_i[b]).wait()
                pltpu.async_copy(g_hbm.at[idx[b].at[0]], rows[b], sem_g[b])

            @pl.when((i >= 2) & (i <= ncw + 1))
            def _():
                pltpu.make_async_copy(g_hbm.at[idx[c].at[0]], rows[c],
                                      sem_g[c]).wait()
                pltpu.async_copy(rows[c], acc.at[idx[c].at[1]], sem_s[c],
                                 add=True)

        return 0

    lax.fori_loop(0, (ncw + 5) // 3, outer, 0)
    plsc.subcore_barrier()
    pltpu.sync_copy(acc.at[pl.ds(sid * RSTRIDE, RWIN)],
                    out_hbm.at[cid, pl.ds(sid * RSTRIDE, RWIN)])


def _dis_inv(degp):
    deg = degp[0, :, 0] + degp[1, :, 0] + 1.0
    return lax.rsqrt(deg), 1.0 / deg


def _gelu(x):
    return 0.5 * x * (1.0 + lax.erf(x * (2.0 ** -0.5)))


def _tc1_body(x_ref, w_ref, degp_ref, h_ref, g_ref):
    dis, _ = _dis_inv(degp_ref[...])
    h = jnp.dot(x_ref[...], w_ref[...], preferred_element_type=jnp.float32)
    h_ref[...] = h
    g_ref[...] = h * dis[:, None]


def _tc_mid_body(sp_ref, hp_ref, degp_ref, b_ref, w_ref, h_ref, g_ref):
    dis, inv = _dis_inv(degp_ref[...])
    s = sp_ref[0] + sp_ref[1]
    pre = s * dis[:, None] + hp_ref[...] * inv[:, None] + b_ref[...]
    a = _gelu(pre)
    h = jnp.dot(a, w_ref[...], preferred_element_type=jnp.float32)
    h_ref[...] = h
    g_ref[...] = h * dis[:, None]


def _tc_fin_body(sp_ref, hp_ref, degp_ref, b_ref, out_ref):
    dis, inv = _dis_inv(degp_ref[...])
    s = sp_ref[0] + sp_ref[1]
    out_ref[...] = s * dis[:, None] + hp_ref[...] * inv[:, None] + b_ref[...]


_ROW_SPEC = pl.BlockSpec((BLK, D), lambda i: (i, 0))
_PART_SPEC = pl.BlockSpec((NC, BLK, D), lambda i: (0, i, 0))
_DEGP_SPEC = pl.BlockSpec((NC, BLK, DEGW), lambda i: (0, i, 0))
_W_SPEC = pl.BlockSpec((D, D), lambda i: (0, 0))
_B_SPEC = pl.BlockSpec((1, D), lambda i: (0, 0))
_ND_F32 = jax.ShapeDtypeStruct((N, D), jnp.float32)


def _tc1(x, w, degp):
    return pl.pallas_call(
        _tc1_body,
        grid=(NBLK,),
        in_specs=[_ROW_SPEC, _W_SPEC, _DEGP_SPEC],
        out_specs=[_ROW_SPEC, _ROW_SPEC],
        out_shape=[_ND_F32, _ND_F32],
    )(x, w, degp)


def _tc_mid(sp, hp, degp, b, w):
    return pl.pallas_call(
        _tc_mid_body,
        grid=(NBLK,),
        in_specs=[_PART_SPEC, _ROW_SPEC, _DEGP_SPEC, _B_SPEC, _W_SPEC],
        out_specs=[_ROW_SPEC, _ROW_SPEC],
        out_shape=[_ND_F32, _ND_F32],
    )(sp, hp, degp, b.reshape(1, D), w)


def _tc_fin(sp, hp, degp, b):
    return pl.pallas_call(
        _tc_fin_body,
        grid=(NBLK,),
        in_specs=[_PART_SPEC, _ROW_SPEC, _DEGP_SPEC, _B_SPEC],
        out_specs=_ROW_SPEC,
        out_shape=_ND_F32,
    )(sp, hp, degp, b.reshape(1, D))


def kernel(x_piece, edge_index_piece, batch, W1, b1, W2, b2, W3, b3):
    ei = edge_index_piece.astype(jnp.int32)

    deg_k = _build_deg_kernel()
    scat_k = _build_scatter_kernel()
    degp = deg_k(ei)

    h1, g1 = _tc1(x_piece, W1, degp)
    s1 = scat_k(g1, ei)
    h2, g2 = _tc_mid(s1, h1, degp, b1, W2)
    s2 = scat_k(g2, ei)
    h3, g3 = _tc_mid(s2, h2, degp, b2, W3)
    s3 = scat_k(g3, ei)
    return _tc_fin(s3, h3, degp, b3)


# speedup vs baseline: 30.4088x; 1.0016x over previous
import functools

import jax
import jax.numpy as jnp
from jax import lax
from jax.experimental import pallas as pl
from jax.experimental.pallas import tpu as pltpu
from jax.experimental.pallas import tpu_sc as plsc

N = 10000
E = 320000
D = 128

NC = 2
NS = 16
NW = NC * NS
K = 128
NCH = E // K
CHW_LO = NCH // NW
CHW_EXTRA = NCH - CHW_LO * NW

RSTRIDE = 624
RWIN = 640
ZR = 128
DEGW = 16

BLK = 1000
NBLK = N // BLK

_SC_MESH = dict(core_axis_name="c", subcore_axis_name="s",
                num_cores=NC, num_subcores=NS)


def _worker_chunks(wid):
    extra = jnp.minimum(wid, CHW_EXTRA)
    start = wid * CHW_LO + extra
    n = jnp.where(wid < CHW_EXTRA, CHW_LO + 1, CHW_LO)
    return start, n


def _zero_rows(zbuf, nrows, width):
    z16 = jnp.zeros((16,), jnp.float32)

    def body(i, _):
        for j in range(width // 16):
            zbuf[i, pl.ds(j * 16, 16)] = z16
        return 0

    lax.fori_loop(0, nrows, body, 0)


@functools.cache
def _build_deg_kernel():
    return pl.kernel(
        _deg_body,
        out_type=jax.ShapeDtypeStruct((NC, N, DEGW), jnp.float32),
        mesh=plsc.VectorSubcoreMesh(**_SC_MESH),
        scratch_types=[
            pltpu.VMEM_SHARED((N, DEGW), jnp.float32),
            pltpu.VMEM((2, K), jnp.int32),
            pltpu.VMEM((2, K), jnp.int32),
            pltpu.VMEM((2, K), jnp.int32),
            pltpu.VMEM((K, DEGW), jnp.float32),
            pltpu.VMEM((ZR, DEGW), jnp.float32),
            pltpu.SemaphoreType.DMA,
            pltpu.SemaphoreType.DMA,
            pltpu.SemaphoreType.DMA,
            pltpu.SemaphoreType.DMA,
            pltpu.SemaphoreType.DMA,
            pltpu.SemaphoreType.DMA,
        ],
    )


def _deg_body(ei_hbm, out_hbm, dacc, i0, i1, i2, ones_v, zbuf,
              si0, si1, si2, ss0, ss1, ss2):
    cid = lax.axis_index("c")
    sid = lax.axis_index("s")
    wid = sid * NC + cid
    ch0, ncw = _worker_chunks(wid)

    idx = (i0, i1, i2)
    sem_i = (si0, si1, si2)
    sem_s = (ss0, ss1, ss2)

    one16 = jnp.ones((16,), jnp.float32)

    def fill(i, _):
        ones_v[i, :] = one16
        return 0

    lax.fori_loop(0, K, fill, 0)
    _zero_rows(zbuf, ZR, DEGW)
    for z in range(RWIN // ZR):
        pltpu.sync_copy(zbuf, dacc.at[pl.ds(sid * RSTRIDE + z * ZR, ZR)])
    plsc.subcore_barrier()

    def outer(t, _):
        for k in range(3):
            i = t * 3 + k
            a, b = k, (k - 1) % 3

            @pl.when((i >= 3) & (i < ncw + 3))
            def _():
                pltpu.make_async_copy(ones_v, dacc.at[idx[a].at[1]],
                                      sem_s[a]).wait()

            @pl.when(i < ncw)
            def _():
                off = (ch0 + i) * K
                pltpu.async_copy(ei_hbm.at[:, pl.ds(off, K)], idx[a],
                                 sem_i[a])

            @pl.when((i >= 1) & (i <= ncw))
            def _():
                pltpu.make_async_copy(ei_hbm.at[:, pl.ds(0, K)], idx[b],
                                      sem_i[b]).wait()
                pltpu.async_copy(ones_v, dacc.at[idx[b].at[1]], sem_s[b],
                                 add=True)

        return 0

    lax.fori_loop(0, (ncw + 5) // 3, outer, 0)
    plsc.subcore_barrier()
    pltpu.sync_copy(dacc.at[pl.ds(sid * RSTRIDE, RWIN)],
                    out_hbm.at[cid, pl.ds(sid * RSTRIDE, RWIN)])


@functools.cache
def _build_scatter_kernel():
    return pl.kernel(
        _scatter_body,
        out_type=jax.ShapeDtypeStruct((NC, N, D), jnp.float32),
        mesh=plsc.VectorSubcoreMesh(**_SC_MESH),
        scratch_types=[
            pltpu.VMEM_SHARED((N, D), jnp.float32),
            pltpu.VMEM((2, K), jnp.int32),
            pltpu.VMEM((2, K), jnp.int32),
            pltpu.VMEM((2, K), jnp.int32),
            pltpu.VMEM((K, D), jnp.float32),
            pltpu.VMEM((K, D), jnp.float32),
            pltpu.VMEM((K, D), jnp.float32),
            pltpu.SemaphoreType.DMA,
            pltpu.SemaphoreType.DMA,
            pltpu.SemaphoreType.DMA,
            pltpu.SemaphoreType.DMA,
            pltpu.SemaphoreType.DMA,
            pltpu.SemaphoreType.DMA,
            pltpu.SemaphoreType.DMA,
            pltpu.SemaphoreType.DMA,
            pltpu.SemaphoreType.DMA,
        ],
    )


def _scatter_body(g_hbm, ei_hbm, out_hbm, acc, i0, i1, i2, r0, r1, r2,
                  si0, si1, si2, sg0, sg1, sg2, ss0, ss1, ss2):
    cid = lax.axis_index("c")
    sid = lax.axis_index("s")
    wid = sid * NC + cid
    ch0, ncw = _worker_chunks(wid)

    idx = (i0, i1, i2)
    rows = (r0, r1, r2)
    sem_i = (si0, si1, si2)
    sem_g = (sg0, sg1, sg2)
    sem_s = (ss0, ss1, ss2)

    _zero_rows(r0, ZR, D)
    for z in range(RWIN // ZR):
        pltpu.sync_copy(r0, acc.at[pl.ds(sid * RSTRIDE + z * ZR, ZR)])
    plsc.subcore_barrier()

    def outer(t, _):
        for k in range(3):
            i = t * 3 + k
            a, b, c = k, (k - 1) % 3, (k - 2) % 3

            @pl.when((i >= 3) & (i < ncw + 3))
            def _():
                pltpu.make_async_copy(rows[a], acc.at[idx[a].at[1]],
                                      sem_s[a]).wait()

            @pl.when(i < ncw)
            def _():
                off = (ch0 + i) * K
                pltpu.async_copy(ei_hbm.at[:, pl.ds(off, K)], idx[a],
                                 sem_i[a])

            @pl.when((i >= 1) & (i <= ncw))
            def _():
                pltpu.make_async_copy(ei_hbm.at[:, pl.ds(0, K)], idx[b],
                                      sem_i[b]).wait()
                pltpu.async_copy(g_hbm.at[idx[b].at[0]], rows[b], sem_g[b])

            @pl.when((i >= 2) & (i <= ncw + 1))
            def _():
                pltpu.make_async_copy(g_hbm.at[idx[c].at[0]], rows[c],
                                      sem_g[c]).wait()
                pltpu.async_copy(rows[c], acc.at[idx[c].at[1]], sem_s[c],
                                 add=True)

        return 0

    lax.fori_loop(0, (ncw + 5) // 3, outer, 0)
    plsc.subcore_barrier()
    pltpu.sync_copy(acc.at[pl.ds(sid * RSTRIDE, RWIN)],
                    out_hbm.at[cid, pl.ds(sid * RSTRIDE, RWIN)])


def _dis_inv(degp):
    deg = degp[0, :, 0] + degp[1, :, 0] + 1.0
    return lax.rsqrt(deg), 1.0 / deg


def _gelu(x):
    return 0.5 * x * (1.0 + lax.erf(x * (2.0 ** -0.5)))


def _m1_body(x_ref, w_ref, h_ref):
    h_ref[...] = jnp.dot(x_ref[...], w_ref[...],
                         preferred_element_type=jnp.float32)


def _s1_body(h_ref, degp_ref, g_ref):
    dis, _ = _dis_inv(degp_ref[...])
    g_ref[...] = h_ref[...] * dis[:, None]


def _tc_mid_body(sp_ref, hp_ref, degp_ref, b_ref, w_ref, h_ref, g_ref):
    dis, inv = _dis_inv(degp_ref[...])
    s = sp_ref[0] + sp_ref[1]
    pre = s * dis[:, None] + hp_ref[...] * inv[:, None] + b_ref[...]
    a = _gelu(pre)
    h = jnp.dot(a, w_ref[...], preferred_element_type=jnp.float32)
    h_ref[...] = h
    g_ref[...] = h * dis[:, None]


def _tc_fin_body(sp_ref, hp_ref, degp_ref, b_ref, out_ref):
    dis, inv = _dis_inv(degp_ref[...])
    s = sp_ref[0] + sp_ref[1]
    out_ref[...] = s * dis[:, None] + hp_ref[...] * inv[:, None] + b_ref[...]


_ROW_SPEC = pl.BlockSpec((BLK, D), lambda i: (i, 0))
_PART_SPEC = pl.BlockSpec((NC, BLK, D), lambda i: (0, i, 0))
_DEGP_SPEC = pl.BlockSpec((NC, BLK, DEGW), lambda i: (0, i, 0))
_W_SPEC = pl.BlockSpec((D, D), lambda i: (0, 0))
_B_SPEC = pl.BlockSpec((1, D), lambda i: (0, 0))
_ND_F32 = jax.ShapeDtypeStruct((N, D), jnp.float32)


def _m1(x, w):
    return pl.pallas_call(
        _m1_body,
        grid=(NBLK,),
        in_specs=[_ROW_SPEC, _W_SPEC],
        out_specs=_ROW_SPEC,
        out_shape=_ND_F32,
    )(x, w)


def _s1(h, degp):
    return pl.pallas_call(
        _s1_body,
        grid=(NBLK,),
        in_specs=[_ROW_SPEC, _DEGP_SPEC],
        out_specs=_ROW_SPEC,
        out_shape=_ND_F32,
    )(h, degp)


def _tc_mid(sp, hp, degp, b, w):
    return pl.pallas_call(
        _tc_mid_body,
        grid=(NBLK,),
        in_specs=[_PART_SPEC, _ROW_SPEC, _DEGP_SPEC, _B_SPEC, _W_SPEC],
        out_specs=[_ROW_SPEC, _ROW_SPEC],
        out_shape=[_ND_F32, _ND_F32],
    )(sp, hp, degp, b.reshape(1, D), w)


def _tc_fin(sp, hp, degp, b):
    return pl.pallas_call(
        _tc_fin_body,
        grid=(NBLK,),
        in_specs=[_PART_SPEC, _ROW_SPEC, _DEGP_SPEC, _B_SPEC],
        out_specs=_ROW_SPEC,
        out_shape=_ND_F32,
    )(sp, hp, degp, b.reshape(1, D))


def kernel(x_piece, edge_index_piece, batch, W1, b1, W2, b2, W3, b3):
    ei = edge_index_piece.astype(jnp.int32)

    deg_k = _build_deg_kernel()
    scat_k = _build_scatter_kernel()
    degp = deg_k(ei)

    h1 = _m1(x_piece, W1)
    g1 = _s1(h1, degp)
    s1 = scat_k(g1, ei)
    h2, g2 = _tc_mid(s1, h1, degp, b1, W2)
    s2 = scat_k(g2, ei)
    h3, g3 = _tc_mid(s2, h2, degp, b2, W3)
    s3 = scat_k(g3, ei)
    return _tc_fin(s3, h3, degp, b3)
